# baseline (speedup 1.0000x reference)
import functools
import math

import jax
import jax.numpy as jnp
from jax import lax
from jax.experimental import pallas as pl
from jax.experimental.pallas import tpu as pltpu


def _cumsum_sublane(m):
    B, E = m.shape
    s = m
    k = 1
    while k < B:
        z = jnp.zeros((k, E), dtype=m.dtype)
        s = s + jnp.concatenate([z, s[:B - k, :]], axis=0)
        k *= 2
    return s


def _moe_block_kernel(x_ref, wg_ref, wcat_ref, sel_ref, fold_ref, be_ref,
                      out_ref, cnt_ref, *, capacity: int, n_experts: int):
    i = pl.program_id(0)

    @pl.when(i == 0)
    def _init():
        cnt_ref[...] = jnp.zeros_like(cnt_ref)

    x = x_ref[...]
    B = x.shape[0]
    E = n_experts

    logits = lax.dot(x, wg_ref[...],
                     preferred_element_type=jnp.float32)
    m = jnp.max(logits, axis=1, keepdims=True)
    p = jnp.exp(logits - m)
    gates = p / jnp.sum(p, axis=1, keepdims=True)

    iota_e = lax.broadcasted_iota(jnp.int32, (B, E), 1)
    is_max = logits == m
    first_max = jnp.min(jnp.where(is_max, iota_e, E), axis=1, keepdims=True)
    mask = (iota_e == first_max).astype(jnp.float32)

    csum = _cumsum_sublane(mask)
    cnt = cnt_ref[...]
    loc = csum - 1.0 + cnt
    keep_mask = mask * (loc < capacity).astype(jnp.float32)
    cnt_ref[...] = cnt + csum[B - 1:B, :]

    coef = gates * keep_mask

    y_all = lax.dot(x, wcat_ref[...],
                    preferred_element_type=jnp.float32)
    coef_b = lax.dot(coef, sel_ref[...], precision=lax.Precision.HIGHEST,
                     preferred_element_type=jnp.float32)
    z = coef_b * (y_all + be_ref[...])
    out_ref[...] = lax.dot(z, fold_ref[...], precision=lax.Precision.HIGHEST,
                           preferred_element_type=jnp.float32)


def kernel(inputs, Wg, We, be):
    d = inputs.shape[-1]
    E = Wg.shape[1]
    x = inputs.reshape(-1, d)
    T = x.shape[0]
    capacity = int(math.ceil(T / E))

    B = 2048
    assert T % B == 0
    n_blocks = T // B

    wcat = We.transpose(1, 0, 2).reshape(d, E * d)
    eye_d = jnp.eye(d, dtype=jnp.float32)
    sel = jnp.repeat(jnp.eye(E, dtype=jnp.float32), d, axis=1)
    fold = jnp.tile(eye_d, (E, 1))
    be_flat = be.reshape(1, E * d)

    out = pl.pallas_call(
        functools.partial(_moe_block_kernel, capacity=capacity, n_experts=E),
        grid=(n_blocks,),
        in_specs=[
            pl.BlockSpec((B, d), lambda i: (i, 0)),
            pl.BlockSpec((d, E), lambda i: (0, 0)),
            pl.BlockSpec((d, E * d), lambda i: (0, 0)),
            pl.BlockSpec((E, E * d), lambda i: (0, 0)),
            pl.BlockSpec((E * d, d), lambda i: (0, 0)),
            pl.BlockSpec((1, E * d), lambda i: (0, 0)),
        ],
        out_specs=pl.BlockSpec((B, d), lambda i: (i, 0)),
        out_shape=jax.ShapeDtypeStruct((T, d), jnp.float32),
        scratch_shapes=[pltpu.VMEM((1, E), jnp.float32)],
        compiler_params=pltpu.CompilerParams(
            dimension_semantics=("arbitrary",)),
    )(x, Wg, wcat, sel, fold, be_flat)
    return out.reshape(inputs.shape)


# speedup vs baseline: 1.6246x; 1.6246x over previous
import functools
import math

import jax
import jax.numpy as jnp
from jax import lax
from jax.experimental import pallas as pl
from jax.experimental.pallas import tpu as pltpu


def _cumsum_sublane(m):
    B, E = m.shape
    s = m
    k = 1
    while k < B:
        z = jnp.zeros((k, E), dtype=m.dtype)
        s = s + jnp.concatenate([z, s[:B - k, :]], axis=0)
        k *= 2
    return s


def _moe_block_kernel(x_ref, wg_ref, wcat_ref, sel_ref, fold_ref, be_ref,
                      out_ref, cnt_ref, *, capacity: int, n_experts: int):
    i = pl.program_id(0)

    @pl.when(i == 0)
    def _init():
        cnt_ref[...] = jnp.zeros_like(cnt_ref)

    x = x_ref[...]
    B = x.shape[0]
    E = n_experts

    logits = lax.dot(x, wg_ref[...],
                     preferred_element_type=jnp.float32)
    m = jnp.max(logits, axis=1, keepdims=True)
    p = jnp.exp(logits - m)
    gates = p / jnp.sum(p, axis=1, keepdims=True)

    iota_e = lax.broadcasted_iota(jnp.int32, (B, E), 1)
    is_max = logits == m
    first_max = jnp.min(jnp.where(is_max, iota_e, E), axis=1, keepdims=True)
    mask = (iota_e == first_max).astype(jnp.float32)

    csum = _cumsum_sublane(mask)
    cnt = cnt_ref[...]
    loc = csum - 1.0 + cnt
    keep_mask = mask * (loc < capacity).astype(jnp.float32)
    cnt_ref[...] = cnt + csum[B - 1:B, :]

    coef = gates * keep_mask

    y_all = lax.dot(x, wcat_ref[...],
                    preferred_element_type=jnp.float32)
    coef_b = lax.dot(coef, sel_ref[...],
                     preferred_element_type=jnp.float32)
    z = coef_b * (y_all + be_ref[...])
    out_ref[...] = lax.dot(z, fold_ref[...],
                           preferred_element_type=jnp.float32)


def kernel(inputs, Wg, We, be):
    d = inputs.shape[-1]
    E = Wg.shape[1]
    x = inputs.reshape(-1, d)
    T = x.shape[0]
    capacity = int(math.ceil(T / E))

    B = 4096
    assert T % B == 0
    n_blocks = T // B

    wcat = We.transpose(1, 0, 2).reshape(d, E * d)
    eye_d = jnp.eye(d, dtype=jnp.float32)
    sel = jnp.repeat(jnp.eye(E, dtype=jnp.float32), d, axis=1)
    fold = jnp.tile(eye_d, (E, 1))
    be_flat = be.reshape(1, E * d)

    out = pl.pallas_call(
        functools.partial(_moe_block_kernel, capacity=capacity, n_experts=E),
        grid=(n_blocks,),
        in_specs=[
            pl.BlockSpec((B, d), lambda i: (i, 0)),
            pl.BlockSpec((d, E), lambda i: (0, 0)),
            pl.BlockSpec((d, E * d), lambda i: (0, 0)),
            pl.BlockSpec((E, E * d), lambda i: (0, 0)),
            pl.BlockSpec((E * d, d), lambda i: (0, 0)),
            pl.BlockSpec((1, E * d), lambda i: (0, 0)),
        ],
        out_specs=pl.BlockSpec((B, d), lambda i: (i, 0)),
        out_shape=jax.ShapeDtypeStruct((T, d), jnp.float32),
        scratch_shapes=[pltpu.VMEM((1, E), jnp.float32)],
        compiler_params=pltpu.CompilerParams(
            dimension_semantics=("arbitrary",)),
    )(x, Wg, wcat, sel, fold, be_flat)
    return out.reshape(inputs.shape)
